# Initial kernel scaffold; baseline (speedup 1.0000x reference)
import jax
import jax.numpy as jnp
from jax.experimental import pallas as pl


def kernel(x, edge_index, edge_attr, batch, params):
    raise NotImplementedError("write your pallas kernel here")


# speedup vs baseline: 1.0072x; 1.0072x over previous
import math
import functools
import jax
import jax.numpy as jnp
from jax import lax
from jax.experimental import pallas as pl
from jax.experimental.pallas import tpu as pltpu

N = 10000
E = 320000
F_IN = 128
ED = 16
DIM = 256
NL = 3
NC = 3
HEADS = 4
SEEDS = 75


def _mm_relu_body(x_ref, w_ref, b_ref, o_ref):
    o_ref[...] = jax.nn.relu(
        jnp.dot(x_ref[...], w_ref[...], preferred_element_type=jnp.float32)
        + b_ref[...]
    )


def _mm_relu(x, w, b, block=1000):
    n, f = x.shape
    d = w.shape[1]
    return pl.pallas_call(
        _mm_relu_body,
        grid=(n // block,),
        in_specs=[
            pl.BlockSpec((block, f), lambda i: (i, 0)),
            pl.BlockSpec((f, d), lambda i: (0, 0)),
            pl.BlockSpec((1, d), lambda i: (0, 0)),
        ],
        out_specs=pl.BlockSpec((block, d), lambda i: (i, 0)),
        out_shape=jax.ShapeDtypeStruct((n, d), jnp.float32),
    )(x, w, b.reshape(1, d))


def _gatv2(h, src, dst, ea, Wl, bl, Wr, br, We, att, bias):
    n = h.shape[0]
    xl = h @ Wl + bl
    xr = h @ Wr + br
    cnt = jax.ops.segment_sum(jnp.ones((ea.shape[0],), jnp.float32), dst, num_segments=n)
    s = jax.ops.segment_sum(ea, dst, num_segments=n)
    loop_attr = s / jnp.maximum(cnt, 1.0)[:, None]
    loops = jnp.arange(n, dtype=src.dtype)
    src_f = jnp.concatenate([src, loops])
    dst_f = jnp.concatenate([dst, loops])
    ea_f = jnp.concatenate([ea, loop_attr], axis=0)
    m = xl[src_f] + xr[dst_f] + ea_f @ We
    m = jax.nn.leaky_relu(m, negative_slope=0.2)
    logit = (m * att).sum(axis=-1)
    lmax = jax.ops.segment_max(logit, dst_f, num_segments=n)
    ex = jnp.exp(logit - lmax[dst_f])
    den = jax.ops.segment_sum(ex, dst_f, num_segments=n)
    alpha = ex / den[dst_f]
    return jax.ops.segment_sum(alpha[:, None] * xl[src_f], dst_f, num_segments=n) + bias


def _gcn(h, src, dst, W, b):
    n = h.shape[0]
    loops = jnp.arange(n, dtype=src.dtype)
    src_f = jnp.concatenate([src, loops])
    dst_f = jnp.concatenate([dst, loops])
    deg = jax.ops.segment_sum(jnp.ones((src_f.shape[0],), jnp.float32), dst_f, num_segments=n)
    dis = jax.lax.rsqrt(jnp.maximum(deg, 1.0))
    norm = dis[src_f] * dis[dst_f]
    hw = h @ W
    return jax.ops.segment_sum(norm[:, None] * hw[src_f], dst_f, num_segments=n) + b


def _mab(Q, K, V, fcq_W, fcq_b, fco_W, fco_b):
    Qp = Q @ fcq_W + fcq_b
    Q_ = jnp.concatenate(jnp.split(Qp, HEADS, axis=2), axis=0)
    K_ = jnp.concatenate(jnp.split(K, HEADS, axis=2), axis=0)
    V_ = jnp.concatenate(jnp.split(V, HEADS, axis=2), axis=0)
    A = jax.nn.softmax(jnp.einsum('bqd,bkd->bqk', Q_, K_) / math.sqrt(DIM), axis=-1)
    O = Q_ + jnp.einsum('bqk,bkd->bqd', A, V_)
    out = jnp.concatenate(jnp.split(O, HEADS, axis=0), axis=2)
    return out + jax.nn.relu(out @ fco_W + fco_b)


def kernel(x, edge_index, edge_attr, batch, params):
    p = params
    src, dst = edge_index[0], edge_index[1]
    h = _mm_relu(x, p['W_in'], p['b_in'])
    for l in range(NL):
        h = _mm_relu(h, p['W_lin'][l], p['b_lin'][l])
    for l in range(NC):
        h = _mm_relu(h, p['W_adj'][l], p['b_adj'][l])
        h = jax.nn.relu(_gatv2(h, src, dst, edge_attr,
                               p['gat_Wl'][l], p['gat_bl'][l],
                               p['gat_Wr'][l], p['gat_br'][l],
                               p['gat_We'][l], p['gat_att'][l], p['gat_bias'][l]))
    z = h @ p['p_lin1_W'] + p['p_lin1_b']
    K = _gcn(z, src, dst, p['g_k_W'], p['g_k_b'])[None]
    V = _gcn(z, src, dst, p['g_v_W'], p['g_v_b'])[None]
    bx = _mab(p['S_g'], K, V, p['g_fcq_W'], p['g_fcq_b'], p['g_fco_W'], p['g_fco_b'])
    bx = _mab(bx, bx @ p['s_k_W'] + p['s_k_b'], bx @ p['s_v_W'] + p['s_v_b'],
              p['s_fcq_W'], p['s_fcq_b'], p['s_fco_W'], p['s_fco_b'])
    bx = _mab(p['S_i'], bx @ p['i_k_W'] + p['i_k_b'], bx @ p['i_v_W'] + p['i_v_b'],
              p['i_fcq_W'], p['i_fcq_b'], p['i_fco_W'], p['i_fco_b'])
    out = bx[:, 0, :] @ p['p_lin2_W'] + p['p_lin2_b']
    return out.reshape(-1)
